# Initial kernel scaffold; baseline (speedup 1.0000x reference)
import jax
import jax.numpy as jnp
from jax.experimental import pallas as pl


def kernel(x, edge_index, edge_attr, params):
    raise NotImplementedError("write your pallas kernel here")


# speedup vs baseline: 2.6014x; 2.6014x over previous
import functools

import jax
import jax.numpy as jnp
from jax import lax
from jax.experimental import pallas as pl
from jax.experimental.pallas import tpu as pltpu
from jax.experimental.pallas import tpu_sc as plsc

N = 10000
E = 320000
ND = 128
ED = 16
HID = 40
DP = 48
NC, NS = 2, 16
NW = NC * NS
BPW = E // NW
GCH = 1000
SCH = 80
EPS = 1e-5
F32 = jnp.float32


def _node_prep_body(x_ref, g_ref, b_ref, w1s_ref, w1d_ref, wq_ref, bq_ref,
                    ps_ref, pd_ref, q_ref):
    x = x_ref[...]
    mu = jnp.mean(x, axis=0, keepdims=True)
    xc = x - mu
    var = jnp.mean(xc * xc, axis=0, keepdims=True)
    xb = jnp.maximum(g_ref[...] * xc / jnp.sqrt(var + EPS) + b_ref[...], 0.0)
    ps_ref[...] = jnp.dot(xb, w1s_ref[...], preferred_element_type=F32)
    pd_ref[...] = jnp.dot(xb, w1d_ref[...], preferred_element_type=F32)
    q_ref[...] = jnp.dot(xb, wq_ref[...], preferred_element_type=F32) + bq_ref[...]


def _node_prep(cur_x, gamma, beta, w1s, w1d, wq, bq):
    return pl.pallas_call(
        _node_prep_body,
        out_shape=[
            jax.ShapeDtypeStruct((N, DP), F32),
            jax.ShapeDtypeStruct((N, DP), F32),
            jax.ShapeDtypeStruct((N, HID), F32),
        ],
    )(cur_x, gamma.reshape(1, ND), beta.reshape(1, ND), w1s, w1d, wq,
      bq.reshape(1, HID))


SBLK = 16000
NSB = E // SBLK


def _colstats_body(a_ref, o_ref, acc_ref):
    ph = pl.program_id(0)
    i = pl.program_id(1)

    @pl.when((ph == 0) & (i == 0))
    def _():
        acc_ref[...] = jnp.zeros_like(acc_ref)

    a = a_ref[...]

    @pl.when(ph == 0)
    def _():
        acc_ref[0:1, :] += jnp.sum(a, axis=0, keepdims=True)

    @pl.when(ph == 1)
    def _():
        mu = acc_ref[0:1, :] * (1.0 / E)
        acc_ref[1:2, :] += jnp.sum((a - mu) ** 2, axis=0, keepdims=True)

    @pl.when((ph == 1) & (i == NSB - 1))
    def _():
        o_ref[...] = acc_ref[...] * (1.0 / E)


def _colstats(arr):
    return pl.pallas_call(
        _colstats_body,
        grid=(2, NSB),
        in_specs=[pl.BlockSpec((SBLK, ED), lambda ph, i: (i, 0))],
        out_specs=pl.BlockSpec((2, ED), lambda ph, i: (0, 0)),
        out_shape=jax.ShapeDtypeStruct((2, ED), F32),
        scratch_shapes=[pltpu.VMEM((2, ED), F32)],
    )(arr)


BE = 8000


def _edge_body_mid(g_ref, ea_ref, st_ref, gm_ref, bt_ref, w1e_ref, b1_ref,
                   w2_ref, b2_ref, eo_ref):
    mu = st_ref[0:1, :]
    var = st_ref[1:2, :]
    eb = jnp.maximum(
        gm_ref[...] * (ea_ref[...] - mu) / jnp.sqrt(var + EPS) + bt_ref[...], 0.0)
    c = jnp.dot(eb, w1e_ref[...], preferred_element_type=F32) + b1_ref[...]
    h = jnp.maximum(g_ref[...][:, :HID] + c, 0.0)
    eo_ref[...] = jnp.dot(h, w2_ref[...], preferred_element_type=F32) + b2_ref[...]


def _edge_body_fin(g_ref, ea_ref, st_ref, gm_ref, bt_ref, w1e_ref, b1_ref,
                   w2_ref, b2_ref, eorig_ref, eo_ref, efin_ref):
    mu = st_ref[0:1, :]
    var = st_ref[1:2, :]
    eb = jnp.maximum(
        gm_ref[...] * (ea_ref[...] - mu) / jnp.sqrt(var + EPS) + bt_ref[...], 0.0)
    c = jnp.dot(eb, w1e_ref[...], preferred_element_type=F32) + b1_ref[...]
    h = jnp.maximum(g_ref[...][:, :HID] + c, 0.0)
    eo = jnp.dot(h, w2_ref[...], preferred_element_type=F32) + b2_ref[...]
    eo_ref[...] = eo
    efin_ref[...] = 0.5 * eorig_ref[...] + 0.5 * eo


def _edge_mlp(g, cur_e, stats, gamma_e, beta_e, w1e, b1, w2, b2, eorig):
    blk = lambda c: pl.BlockSpec((BE, c), lambda i: (i, 0))
    fix = lambda r, c: pl.BlockSpec((r, c), lambda i: (0, 0))
    args = [g, cur_e, stats, gamma_e.reshape(1, ED), beta_e.reshape(1, ED),
            w1e, b1.reshape(1, HID), w2, b2.reshape(1, ED)]
    in_specs = [blk(DP), blk(ED), fix(2, ED), fix(1, ED), fix(1, ED),
                fix(ED, HID), fix(1, HID), fix(HID, ED), fix(1, ED)]
    if eorig is None:
        return pl.pallas_call(
            _edge_body_mid,
            grid=(E // BE,),
            in_specs=in_specs,
            out_specs=blk(ED),
            out_shape=jax.ShapeDtypeStruct((E, ED), F32),
        )(*args)
    return pl.pallas_call(
        _edge_body_fin,
        grid=(E // BE,),
        in_specs=in_specs + [blk(ED)],
        out_specs=[blk(ED), blk(ED)],
        out_shape=[jax.ShapeDtypeStruct((E, ED), F32),
                   jax.ShapeDtypeStruct((E, ED), F32)],
    )(*(args + [eorig]))


def _node_body_mid(q_ref, agg_ref, wa_ref, w2_ref, b2_ref, out_ref):
    agg = agg_ref[0] + agg_ref[1]
    t = jnp.maximum(q_ref[...] + jnp.dot(agg, wa_ref[...],
                                         preferred_element_type=F32), 0.0)
    out_ref[...] = jnp.dot(t, w2_ref[...], preferred_element_type=F32) + b2_ref[...]


def _node_body_fin(q_ref, agg_ref, wa_ref, w2_ref, b2_ref, xorig_ref, out_ref):
    agg = agg_ref[0] + agg_ref[1]
    t = jnp.maximum(q_ref[...] + jnp.dot(agg, wa_ref[...],
                                         preferred_element_type=F32), 0.0)
    xo = jnp.dot(t, w2_ref[...], preferred_element_type=F32) + b2_ref[...]
    out_ref[...] = 0.5 * xorig_ref[...] + 0.5 * xo


def _node_mlp(q, agg2, wa, w2, b2, xorig):
    args = [q, agg2, wa, w2, b2.reshape(1, ND)]
    if xorig is None:
        return pl.pallas_call(
            _node_body_mid,
            out_shape=jax.ShapeDtypeStruct((N, ND), F32),
        )(*args)
    return pl.pallas_call(
        _node_body_fin,
        out_shape=jax.ShapeDtypeStruct((N, ND), F32),
    )(*(args + [xorig]))


def _sc_gather(ps, pd, src, dst):
    mesh = plsc.VectorSubcoreMesh(core_axis_name="c", subcore_axis_name="s")

    @functools.partial(
        pl.kernel, mesh=mesh,
        out_type=jax.ShapeDtypeStruct((E, DP), F32),
        compiler_params=pltpu.CompilerParams(use_tc_tiling_on_sc=False),
        scratch_types=[
            pltpu.VMEM((GCH,), jnp.int32),
            pltpu.VMEM((GCH,), jnp.int32),
            pltpu.VMEM((GCH, DP), F32),
            pltpu.VMEM((GCH, DP), F32),
            pltpu.SemaphoreType.DMA,
            pltpu.SemaphoreType.DMA,
        ],
    )
    def k(ps_hbm, pd_hbm, src_hbm, dst_hbm, out_hbm, si, di, rs, rd, s1, s2):
        wid = lax.axis_index("s") * NC + lax.axis_index("c")
        base = wid * BPW

        def chunk(j, carry):
            off = base + j * GCH
            pltpu.sync_copy(src_hbm.at[pl.ds(off, GCH)], si)
            pltpu.sync_copy(dst_hbm.at[pl.ds(off, GCH)], di)
            c1 = pltpu.async_copy(ps_hbm.at[si], rs, s1)
            c2 = pltpu.async_copy(pd_hbm.at[di], rd, s2)
            c1.wait()
            c2.wait()

            def addrow(r, cc):
                for c in range(DP // 16):
                    sl = (r, pl.ds(c * 16, 16))
                    rs[sl] = rs[sl] + rd[sl]
                return cc

            lax.fori_loop(0, GCH, addrow, 0)
            pltpu.sync_copy(rs, out_hbm.at[pl.ds(off, GCH)])
            return carry

        lax.fori_loop(0, BPW // GCH, chunk, 0)

    return k(ps, pd, src, dst)


def _sc_scatter(eo, dst):
    mesh = plsc.VectorSubcoreMesh(core_axis_name="c", subcore_axis_name="s")
    rpt = N // NS

    @functools.partial(
        pl.kernel, mesh=mesh,
        out_type=jax.ShapeDtypeStruct((NC, N, ED), F32),
        compiler_params=pltpu.CompilerParams(use_tc_tiling_on_sc=False),
        scratch_types=[
            pltpu.VMEM((SCH,), jnp.int32),
            pltpu.VMEM((SCH, ED), F32),
            pltpu.VMEM((rpt, ED), F32),
            pltpu.VMEM_SHARED((N, ED), F32),
        ],
    )
    def k(eo_hbm, dst_hbm, out_hbm, idxv, rows, stage, aggsh):
        cid = lax.axis_index("c")
        sid = lax.axis_index("s")
        wid = sid * NC + cid
        base = wid * BPW

        def zrow(r, carry):
            stage[r, pl.ds(0, ED)] = jnp.zeros((ED,), F32)
            return carry

        lax.fori_loop(0, rpt, zrow, 0)
        pltpu.sync_copy(stage, aggsh.at[pl.ds(sid * rpt, rpt)])
        plsc.subcore_barrier()

        def chunk(j, carry):
            off = base + j * SCH
            pltpu.sync_copy(dst_hbm.at[pl.ds(off, SCH)], idxv)
            pltpu.sync_copy(eo_hbm.at[pl.ds(off, SCH)], rows)
            pltpu.sync_copy(rows, aggsh.at[idxv], add=True)
            return carry

        lax.fori_loop(0, BPW // SCH, chunk, 0)
        plsc.subcore_barrier()
        pltpu.sync_copy(aggsh.at[pl.ds(sid * rpt, rpt)], stage)
        pltpu.sync_copy(stage, out_hbm.at[cid, pl.ds(sid * rpt, rpt)])

    return k(eo, dst)


def kernel(x, edge_index, edge_attr, params):
    layers = params["layers"]
    src = edge_index[0]
    dst = edge_index[1]
    cur_x, cur_e = x, edge_attr
    out_x = out_e = None
    for li, p in enumerate(layers):
        final = li == len(layers) - 1
        pe, pn = p["edge_mlp"], p["node_mlp"]
        w1 = pe["W1"]
        pad = ((0, 0), (0, DP - HID))
        w1s = jnp.pad(w1[:ND], pad)
        w1d = jnp.pad(w1[ND:2 * ND], pad)
        w1e = w1[2 * ND:]
        wn1 = pn["W1"]
        ps, pd_, q = _node_prep(cur_x, p["bn_node"]["gamma"], p["bn_node"]["beta"],
                                w1s, w1d, wn1[:ND], pn["b1"])
        stats = _colstats(cur_e)
        g = _sc_gather(ps, pd_, src, dst)
        if final:
            eo, out_e = _edge_mlp(g, cur_e, stats, p["bn_edge"]["gamma"],
                                  p["bn_edge"]["beta"], w1e, pe["b1"], pe["W2"],
                                  pe["b2"], edge_attr)
        else:
            eo = _edge_mlp(g, cur_e, stats, p["bn_edge"]["gamma"],
                           p["bn_edge"]["beta"], w1e, pe["b1"], pe["W2"],
                           pe["b2"], None)
        agg2 = _sc_scatter(eo, dst)
        xo = _node_mlp(q, agg2, wn1[ND:], pn["W2"], pn["b2"],
                       x if final else None)
        if final:
            out_x = xo
        cur_x, cur_e = xo, eo
    return (out_x, out_e)
